# baseline (speedup 1.0000x reference)
import functools

import jax
import jax.numpy as jnp
from jax.experimental import pallas as pl
from jax.experimental.pallas import tpu as pltpu

B = 16384
D = 128
TILE = 16384


def _fused_gate_kernel(mc_ref, m_ref, wg_ref, b_ref, out_ref):
    mc = mc_ref[...]
    m = m_ref[...]
    wg = wg_ref[...]
    b = b_ref[...]
    w1 = wg[:, :D]
    w2 = wg[:, D:]
    v = jax.lax.dot_general(m, w1, (((1,), (1,)), ((), ())),
                            preferred_element_type=jnp.float32) + b
    logits = jax.lax.dot_general(mc, w2, (((1,), (1,)), ((), ())),
                                 preferred_element_type=jnp.float32) + v
    gate = jax.nn.sigmoid(logits)
    out_ref[...] = gate * (m - mc) + mc


@functools.partial(jax.jit, donate_argnums=())
def kernel(current_state, memory_context, W_attn, b_attn, W_gate, b_gate):
    del W_attn, b_attn
    m = jax.lax.stop_gradient(current_state[0:1])
    b2 = b_gate.reshape(1, D)
    grid = (B // TILE,)
    return pl.pallas_call(
        _fused_gate_kernel,
        grid=grid,
        in_specs=[
            pl.BlockSpec((TILE, D), lambda i: (i, 0)),
            pl.BlockSpec((1, D), lambda i: (0, 0)),
            pl.BlockSpec((D, 2 * D), lambda i: (0, 0)),
            pl.BlockSpec((1, D), lambda i: (0, 0)),
        ],
        out_specs=pl.BlockSpec((TILE, D), lambda i: (i, 0)),
        out_shape=jax.ShapeDtypeStruct((B, D), jnp.float32),
        compiler_params=pltpu.CompilerParams(
            dimension_semantics=("arbitrary",),
        ),
    )(memory_context, m, W_gate, b2)


# speedup vs baseline: 1.2542x; 1.2542x over previous
import functools

import jax
import jax.numpy as jnp
from jax.experimental import pallas as pl
from jax.experimental.pallas import tpu as pltpu

B = 16384
D = 128
TILE = 8192


def _fused_gate_kernel(mc_ref, m_ref, wg_ref, b_ref, out_ref):
    mc = mc_ref[...]
    m = m_ref[...]
    wg = wg_ref[...]
    b = b_ref[...]
    w1 = wg[:, :D]
    w2 = wg[:, D:]
    v = jax.lax.dot_general(m, w1, (((1,), (1,)), ((), ())),
                            preferred_element_type=jnp.float32) + b
    logits = jax.lax.dot_general(mc, w2, (((1,), (1,)), ((), ())),
                                 preferred_element_type=jnp.float32) + v
    gate = jax.nn.sigmoid(logits)
    out_ref[...] = gate * (m - mc) + mc


@functools.partial(jax.jit, donate_argnums=())
def kernel(current_state, memory_context, W_attn, b_attn, W_gate, b_gate):
    del W_attn, b_attn
    m = jax.lax.stop_gradient(current_state[0:1])
    b2 = b_gate.reshape(1, D)
    grid = (B // TILE,)
    return pl.pallas_call(
        _fused_gate_kernel,
        grid=grid,
        in_specs=[
            pl.BlockSpec((TILE, D), lambda i: (i, 0)),
            pl.BlockSpec((1, D), lambda i: (0, 0)),
            pl.BlockSpec((D, 2 * D), lambda i: (0, 0)),
            pl.BlockSpec((1, D), lambda i: (0, 0)),
        ],
        out_specs=pl.BlockSpec((TILE, D), lambda i: (i, 0)),
        out_shape=jax.ShapeDtypeStruct((B, D), jnp.float32),
        compiler_params=pltpu.CompilerParams(
            dimension_semantics=("parallel",),
        ),
    )(memory_context, m, W_gate, b2)
